# baseline (speedup 1.0000x reference)
import jax
import jax.numpy as jnp
from jax import lax
from jax.experimental import pallas as pl
from jax.experimental.pallas import tpu as pltpu
from jax.experimental.pallas import tpu_sc as plsc

B, C, H, W = 32, 192, 64, 64
NSEL = 32
NW = 32
L = 16
NB = 8
LD = 4


def _sc_kernel(xp_hbm, ri_hbm, rj_hbm, out_hbm, idx_v, slab_v, gsem, osem):
    nc = 2
    wid = lax.axis_index("s") * nc + lax.axis_index("c")
    b = wid

    pltpu.sync_copy(ri_hbm, idx_v.at[0])
    pltpu.sync_copy(rj_hbm, idx_v.at[1])
    ri_lo = idx_v[0, pl.ds(0, L)]
    ri_hi = idx_v[0, pl.ds(L, L)]
    rj_lo = idx_v[1, pl.ds(0, L)]
    rj_hi = idx_v[1, pl.ds(L, L)]
    lanes = lax.iota(jnp.int32, L)

    def vec_at(lo, hi, i):
        return (jnp.sum(jnp.where(lanes == i, lo, 0))
                + jnp.sum(jnp.where(lanes == i - L, hi, 0)))

    shift1 = jnp.where(lanes == L - 1, 0, lanes + 1)

    def fire_gather(i, k):
        h = vec_at(rj_lo, rj_hi, i)

        def jbody(off):
            def jb(j, v):
                pltpu.async_copy(xp_hbm.at[b, h, v[0]],
                                 slab_v.at[k, j + off], gsem.at[k])
                return lax.gather(
                    v, shift1[:, None],
                    dimension_numbers=lax.GatherDimensionNumbers(
                        offset_dims=(), collapsed_slice_dims=(0,),
                        start_index_map=(0,)),
                    slice_sizes=(1,),
                    mode=lax.GatherScatterMode.PROMISE_IN_BOUNDS)
            return jb

        lax.fori_loop(0, L, jbody(0), ri_lo)
        lax.fori_loop(0, L, jbody(L), ri_hi)

    def drain_gather(k):
        pltpu.make_async_copy(xp_hbm.at[b, 0, pl.ds(0, NSEL)],
                              slab_v.at[k], gsem.at[k]).wait()

    def drain_out(i, k):
        pltpu.make_async_copy(slab_v.at[k], out_hbm.at[b, i],
                              osem.at[k]).wait()

    for q in range(LD):
        fire_gather(q, q)

    def step(t, carry):
        for kk in range(NB):
            g = t * NB + kk
            drain_gather(kk)
            pltpu.async_copy(slab_v.at[kk], out_hbm.at[b, g], osem.at[kk])

            @pl.when(g >= LD)
            def _():
                drain_out(g - LD, (kk + LD) % NB)

            @pl.when(g + LD < NSEL)
            def _():
                fire_gather(g + LD, (kk + LD) % NB)
        return carry

    lax.fori_loop(0, NSEL // NB, step, 0)
    for q in range(NSEL - LD, NSEL):
        drain_out(q, q % NB)


def kernel(x, randi, randj):
    x_p = jnp.transpose(x, (0, 2, 3, 1))
    mesh = plsc.VectorSubcoreMesh(core_axis_name="c", subcore_axis_name="s")
    run = pl.kernel(
        _sc_kernel,
        out_type=jax.ShapeDtypeStruct((B, NSEL, NSEL, C), jnp.float32),
        mesh=mesh,
        compiler_params=pltpu.CompilerParams(needs_layout_passes=False),
        scratch_types=[
            pltpu.VMEM((2, NSEL), jnp.int32),
            pltpu.VMEM((NB, NSEL, C), jnp.float32),
            pltpu.SemaphoreType.DMA((NB,)),
            pltpu.SemaphoreType.DMA((NB,)),
        ],
    )
    out_p = run(x_p, randi, randj)
    return jnp.transpose(out_p, (0, 3, 1, 2))


# speedup vs baseline: 1.0028x; 1.0028x over previous
import jax
import jax.numpy as jnp
from jax import lax
from jax.experimental import pallas as pl
from jax.experimental.pallas import tpu as pltpu
from jax.experimental.pallas import tpu_sc as plsc

B, C, H, W = 32, 192, 64, 64
NSEL = 32
NW = 32
L = 16
NB = 8
LD = 4


def _sc_kernel(xp_hbm, ri_hbm, rj_hbm, out_hbm, idx_v, slab_v, gsem, osem):
    nc = 2
    wid = lax.axis_index("s") * nc + lax.axis_index("c")
    b = wid

    pltpu.sync_copy(ri_hbm, idx_v.at[0])
    pltpu.sync_copy(rj_hbm, idx_v.at[1])
    ri_lo = idx_v[0, pl.ds(0, L)]
    ri_hi = idx_v[0, pl.ds(L, L)]
    rj_lo = idx_v[1, pl.ds(0, L)]
    rj_hi = idx_v[1, pl.ds(L, L)]
    lanes = lax.iota(jnp.int32, L)

    def vec_at(lo, hi, i):
        return (jnp.sum(jnp.where(lanes == i, lo, 0))
                + jnp.sum(jnp.where(lanes == i - L, hi, 0)))

    def fire_gather(i, k):
        h = vec_at(rj_lo, rj_hi, i)

        def jbody(j, carry):
            w = vec_at(ri_lo, ri_hi, j)
            pltpu.async_copy(xp_hbm.at[b, h, w], slab_v.at[k, j],
                             gsem.at[k])
            return carry

        lax.fori_loop(0, NSEL, jbody, 0)

    def drain_gather(k):
        pltpu.make_async_copy(xp_hbm.at[b, 0, pl.ds(0, NSEL)],
                              slab_v.at[k], gsem.at[k]).wait()

    def drain_out(i, k):
        pltpu.make_async_copy(slab_v.at[k], out_hbm.at[b, i],
                              osem.at[k]).wait()

    for q in range(LD):
        fire_gather(q, q)

    def step(t, carry):
        for kk in range(NB):
            g = t * NB + kk
            drain_gather(kk)
            pltpu.async_copy(slab_v.at[kk], out_hbm.at[b, g], osem.at[kk])

            @pl.when(g >= LD)
            def _():
                drain_out(g - LD, (kk + LD) % NB)

            @pl.when(g + LD < NSEL)
            def _():
                fire_gather(g + LD, (kk + LD) % NB)
        return carry

    lax.fori_loop(0, NSEL // NB, step, 0)
    for q in range(NSEL - LD, NSEL):
        drain_out(q, q % NB)


def kernel(x, randi, randj):
    x_p = jnp.transpose(x, (0, 2, 3, 1))
    mesh = plsc.VectorSubcoreMesh(core_axis_name="c", subcore_axis_name="s")
    run = pl.kernel(
        _sc_kernel,
        out_type=jax.ShapeDtypeStruct((B, NSEL, NSEL, C), jnp.float32),
        mesh=mesh,
        compiler_params=pltpu.CompilerParams(needs_layout_passes=False),
        scratch_types=[
            pltpu.VMEM((2, NSEL), jnp.int32),
            pltpu.VMEM((NB, NSEL, C), jnp.float32),
            pltpu.SemaphoreType.DMA((NB,)),
            pltpu.SemaphoreType.DMA((NB,)),
        ],
    )
    out_p = run(x_p, randi, randj)
    return jnp.transpose(out_p, (0, 3, 1, 2))
